# Initial kernel scaffold; baseline (speedup 1.0000x reference)
import jax
import jax.numpy as jnp
from jax.experimental import pallas as pl


def kernel(z_a, z_b, z_a_local, z_b_local, location_a, location_b):
    raise NotImplementedError("write your pallas kernel here")


# speedup vs baseline: 4.1591x; 4.1591x over previous
import jax
import jax.numpy as jnp
from jax import lax
from jax.experimental import pallas as pl

_LAMBDA = 25.0
_MU = 25.0
_NU = 1.0
_ALPHA = 0.25
_EPS = 1e-4
_K0, _K1 = 20, 4


def _local_stats_kernel(za_ref, zb_ref, la_ref, lb_ref,
                        rmf_ref, rml_ref, ga_ref, cmf_ref, cml_ref, gb_ref):
    za = za_ref[0]
    zb = zb_ref[0]
    la = la_ref[0]
    lb = lb_ref[0]
    n = za.shape[0]

    ones_c = jnp.ones((1, za.shape[1]), jnp.float32)
    ones_2 = jnp.ones((1, la.shape[1]), jnp.float32)

    na = jnp.sum(za * za, axis=1, keepdims=True)
    nb_t = lax.dot_general(ones_c, zb * zb, (((1,), (1,)), ((), ())),
                           preferred_element_type=jnp.float32)
    cross = lax.dot_general(za, zb, (((1,), (1,)), ((), ())),
                            preferred_element_type=jnp.float32)
    d2f = jnp.maximum(na + nb_t - 2.0 * cross, 0.0)

    nla = jnp.sum(la * la, axis=1, keepdims=True)
    nlb_t = lax.dot_general(ones_2, lb * lb, (((1,), (1,)), ((), ())),
                            preferred_element_type=jnp.float32)
    lcross = lax.dot_general(la, lb, (((1,), (1,)), ((), ())),
                             preferred_element_type=jnp.float32)
    d2l = jnp.maximum(nla + nlb_t - 2.0 * lcross, 0.0)

    iota_j = lax.broadcasted_iota(jnp.int32, (n, n), 1)
    iota_i = lax.broadcasted_iota(jnp.int32, (n, n), 0)

    rmf = jnp.min(d2f, axis=1, keepdims=True)
    cmf = jnp.min(d2f, axis=0, keepdims=True)
    rml = jnp.min(d2l, axis=1, keepdims=True)
    cml = jnp.min(d2l, axis=0, keepdims=True)
    ral = jnp.min(jnp.where(d2l == rml, iota_j, n), axis=1, keepdims=True)
    cal = jnp.min(jnp.where(d2l == cml, iota_i, n), axis=0, keepdims=True)
    ga = jnp.sum(jnp.where(iota_j == ral, d2f, 0.0), axis=1, keepdims=True)
    gb = jnp.sum(jnp.where(iota_i == cal, d2f, 0.0), axis=0, keepdims=True)

    rmf_ref[0] = rmf
    rml_ref[0] = rml
    ga_ref[0] = ga
    cmf_ref[0] = cmf
    cml_ref[0] = cml
    gb_ref[0] = gb


def _topk_sum(keys, vals, k):
    b, n = keys.shape
    iota = lax.broadcasted_iota(jnp.int32, (b, n), 1)
    total = jnp.zeros((b, 1), jnp.float32)
    for _ in range(k):
        m = jnp.min(keys, axis=1, keepdims=True)
        idx = jnp.min(jnp.where(keys == m, iota, n), axis=1, keepdims=True)
        onehot = iota == idx
        total = total + jnp.sum(jnp.where(onehot, vals, 0.0), axis=1,
                                keepdims=True)
        keys = jnp.where(onehot, jnp.float32(jnp.inf), keys)
    return jnp.sum(total)


def _finalize_kernel(za_ref, zb_ref, rmf_ref, cmf_ref, rml_ref, cml_ref,
                     ga_ref, gb_ref, o_ref):
    za = za_ref[...]
    zb = zb_ref[...]
    b, d = za.shape
    bn = float(b)
    nm1 = bn - 1.0

    def global_stats(x):
        mean0 = jnp.mean(x, axis=0, keepdims=True)
        xc = x - mean0
        s = jnp.sum(xc * xc, axis=0, keepdims=True)
        g = lax.dot_general(xc, xc, (((1,), (1,)), ((), ())),
                            preferred_element_type=jnp.float32)
        cov = (jnp.sum(g * g) - jnp.sum(s * s)) / (nm1 * nm1 * d)
        std = jnp.sqrt(s / nm1 + _EPS)
        var = jnp.mean(jnp.maximum(1.0 - std, 0.0))
        return var, cov

    var_a, cov_a = global_stats(za)
    var_b, cov_b = global_stats(zb)
    diff = za - zb
    inv_g = jnp.mean(diff * diff)
    g_loss = (_LAMBDA * inv_g + _MU * 0.5 * (var_a + var_b)
              + _NU * (cov_a + cov_b))

    rmf = rmf_ref[...]
    cmf = cmf_ref[...]
    rml = rml_ref[...]
    cml = cml_ref[...]
    ga = ga_ref[...]
    gb = gb_ref[...]
    nb = rmf.shape[0]
    c = 768.0

    s0 = _topk_sum(rmf, rmf, _K0)
    s1 = _topk_sum(cmf, cmf, _K1)
    s2 = _topk_sum(rml, ga, _K0)
    s3 = _topk_sum(cml, gb, _K1)

    inv_l = (s0 / (2.0 * nb * _K0 * c) + s1 / (2.0 * nb * _K1 * c)
             + s2 / (2.0 * nb * _K0 * c) + s3 / (2.0 * nb * _K1 * c))
    l_loss = _LAMBDA * inv_l

    out = _ALPHA * g_loss + (1.0 - _ALPHA) * l_loss
    o_ref[...] = jnp.broadcast_to(out, (1, 1))


def kernel(z_a, z_b, z_a_local, z_b_local, location_a, location_b,
           interpret=False):
    b, h, w, c = z_a_local.shape
    n = h * w
    za = z_a_local.reshape(b, n, c)
    zb = z_b_local.reshape(b, n, c)
    la = location_a.reshape(b, n, 2)
    lb = location_b.reshape(b, n, 2)

    col = pl.BlockSpec((1, n, 1), lambda i: (i, 0, 0))
    row = pl.BlockSpec((1, 1, n), lambda i: (i, 0, 0))
    rmf, rml, ga, cmf, cml, gb = pl.pallas_call(
        _local_stats_kernel,
        grid=(b,),
        in_specs=[pl.BlockSpec((1, n, c), lambda i: (i, 0, 0)),
                  pl.BlockSpec((1, n, c), lambda i: (i, 0, 0)),
                  pl.BlockSpec((1, n, 2), lambda i: (i, 0, 0)),
                  pl.BlockSpec((1, n, 2), lambda i: (i, 0, 0))],
        out_specs=[col, col, col, row, row, row],
        out_shape=[jax.ShapeDtypeStruct((b, n, 1), jnp.float32)] * 3
                  + [jax.ShapeDtypeStruct((b, 1, n), jnp.float32)] * 3,
        interpret=interpret,
    )(za, zb, la, lb)

    out = pl.pallas_call(
        _finalize_kernel,
        out_shape=jax.ShapeDtypeStruct((1, 1), jnp.float32),
        interpret=interpret,
    )(z_a, z_b, rmf.reshape(b, n), cmf.reshape(b, n), rml.reshape(b, n),
      cml.reshape(b, n), ga.reshape(b, n), gb.reshape(b, n))
    return out.reshape(())
